# baseline (speedup 1.0000x reference)
import functools

import jax
import jax.numpy as jnp
from jax import lax
from jax.experimental import pallas as pl
from jax.experimental.pallas import tpu as pltpu
from jax.experimental.pallas import tpu_sc as plsc

N = 10000
D = 128
NNZ = 320000
NC, NS, L = 2, 16, 16
NW = NC * NS
NNZ_PAD = 327680
NP = 10240
ROWS_PER_TILE = NP // NS
GSZ = 128
GD = 80
SG = 64
SGN = 160
NBUF = 2


def _sc_mesh():
    return plsc.VectorSubcoreMesh(core_axis_name="c", subcore_axis_name="s")


def _degrees(v_blk, e_blk):
    @functools.partial(
        pl.kernel,
        out_type=(
            jax.ShapeDtypeStruct((NC, NS, ROWS_PER_TILE), jnp.float32),
            jax.ShapeDtypeStruct((NC, NS, ROWS_PER_TILE), jnp.float32),
        ),
        mesh=_sc_mesh(),
        scratch_types=[
            pltpu.VMEM((GD, GSZ), jnp.int32),
            pltpu.VMEM((GD, GSZ), jnp.int32),
            pltpu.VMEM((GSZ,), jnp.float32),
            pltpu.VMEM((ROWS_PER_TILE,), jnp.float32),
            pltpu.VMEM_SHARED((NP,), jnp.float32),
            pltpu.VMEM_SHARED((NP,), jnp.float32),
        ],
    )
    def k(v_hbm, e_hbm, degv_hbm, dege_hbm, idx_v, idx_e, ones, zbuf,
          accv, acce):
        c = lax.axis_index("c")
        s = lax.axis_index("s")
        wid = s * NC + c

        def fill(i, _):
            ones[pl.ds(i * L, L)] = jnp.ones((L,), jnp.float32)
            return 0

        lax.fori_loop(0, GSZ // L, fill, 0)

        def zfill(i, _):
            zbuf[pl.ds(i * L, L)] = jnp.zeros((L,), jnp.float32)
            return 0

        lax.fori_loop(0, ROWS_PER_TILE // L, zfill, 0)

        base = s * ROWS_PER_TILE
        pltpu.sync_copy(zbuf, accv.at[pl.ds(base, ROWS_PER_TILE)])
        pltpu.sync_copy(zbuf, acce.at[pl.ds(base, ROWS_PER_TILE)])
        pltpu.sync_copy(v_hbm.at[wid], idx_v)
        pltpu.sync_copy(e_hbm.at[wid], idx_e)
        plsc.subcore_barrier()

        def body(g, _):
            pltpu.sync_copy(ones, accv.at[idx_v.at[g]], add=True)
            pltpu.sync_copy(ones, acce.at[idx_e.at[g]], add=True)
            return 0

        lax.fori_loop(0, GD, body, 0)
        plsc.subcore_barrier()

        pltpu.sync_copy(accv.at[pl.ds(base, ROWS_PER_TILE)], degv_hbm.at[c, s])
        pltpu.sync_copy(acce.at[pl.ds(base, ROWS_PER_TILE)], dege_hbm.at[c, s])

    return k(v_blk, e_blk)


def _smooth_phase(table, g_blk, s_blk):
    @functools.partial(
        pl.kernel,
        out_type=jax.ShapeDtypeStruct((NC, NS, ROWS_PER_TILE, D),
                                      jnp.float32),
        mesh=_sc_mesh(),
        compiler_params=pltpu.CompilerParams(use_tc_tiling_on_sc=False),
        scratch_types=[
            pltpu.VMEM((SGN, SG), jnp.int32),
            pltpu.VMEM((SGN, SG), jnp.int32),
            [pltpu.VMEM((SG, D), jnp.float32) for _ in range(NBUF)],
            pltpu.VMEM_SHARED((NP, D), jnp.float32),
            [pltpu.SemaphoreType.DMA for _ in range(NBUF)],
            [pltpu.SemaphoreType.DMA for _ in range(NBUF)],
        ],
    )
    def k(tab, g_hbm, s_hbm, out_hbm, idx_g, idx_s, bufs, acc,
          gsem, ssem):
        c = lax.axis_index("c")
        s = lax.axis_index("s")
        wid = s * NC + c
        base = s * ROWS_PER_TILE

        def zrow(i, _):
            r = i // (D // L)
            col = (i % (D // L)) * L
            bufs[0][r, pl.ds(col, L)] = jnp.zeros((L,), jnp.float32)
            return 0

        lax.fori_loop(0, SG * (D // L), zrow, 0)

        for z in range(ROWS_PER_TILE // SG):
            pltpu.sync_copy(bufs[0], acc.at[pl.ds(base + z * SG, SG)])

        pltpu.sync_copy(g_hbm.at[wid], idx_g)
        pltpu.sync_copy(s_hbm.at[wid], idx_s)
        plsc.subcore_barrier()

        for b in range(NBUF):
            pltpu.async_copy(tab.at[idx_g.at[b]], bufs[b], gsem[b])

        def body(i, _):
            for b in range(NBUF):
                g = NBUF * i + b
                pltpu.make_async_copy(
                    tab.at[idx_g.at[g]], bufs[b], gsem[b]).wait()
                pltpu.async_copy(
                    bufs[b], acc.at[idx_s.at[g]], ssem[b], add=True)

            @pl.when(i + 1 < SGN // NBUF)
            def _():
                for b in range(NBUF):
                    g = NBUF * (i + 1) + b
                    pltpu.make_async_copy(
                        bufs[b], acc.at[idx_s.at[g - NBUF]], ssem[b]).wait()
                    pltpu.async_copy(tab.at[idx_g.at[g]], bufs[b], gsem[b])

            return 0

        lax.fori_loop(0, SGN // NBUF, body, 0)
        for b in range(NBUF):
            pltpu.make_async_copy(
                bufs[b], acc.at[idx_s.at[SGN - NBUF + b]], ssem[b]).wait()
        plsc.subcore_barrier()

        pltpu.sync_copy(acc.at[pl.ds(base, ROWS_PER_TILE)], out_hbm.at[c, s])

    return k(table, g_blk, s_blk)


_BROWS = 1280


def _proj_kernel(x_ref, w_ref, b_ref, dv_ref, de_ref, y_ref, isdv_ref,
                 ide_ref):
    dv = dv_ref[0] + dv_ref[1]
    de = de_ref[0] + de_ref[1]
    isdv = jnp.where(dv > 0, lax.rsqrt(dv), 0.0)
    isdv_ref[...] = isdv
    ide_ref[...] = jnp.where(de > 0, 1.0 / de, 0.0)
    y = jnp.dot(x_ref[...], w_ref[...].T, preferred_element_type=jnp.float32)
    y_ref[...] = (y + b_ref[...]) * isdv


def _project(x_pad, w, b, degv, dege):
    return pl.pallas_call(
        _proj_kernel,
        grid=(NP // _BROWS,),
        in_specs=[
            pl.BlockSpec((_BROWS, D), lambda i: (i, 0)),
            pl.BlockSpec((D, D), lambda i: (0, 0)),
            pl.BlockSpec((1, D), lambda i: (0, 0)),
            pl.BlockSpec((2, _BROWS, 1), lambda i: (0, i, 0)),
            pl.BlockSpec((2, _BROWS, 1), lambda i: (0, i, 0)),
        ],
        out_specs=[
            pl.BlockSpec((_BROWS, D), lambda i: (i, 0)),
            pl.BlockSpec((_BROWS, 1), lambda i: (i, 0)),
            pl.BlockSpec((_BROWS, 1), lambda i: (i, 0)),
        ],
        out_shape=[
            jax.ShapeDtypeStruct((NP, D), jnp.float32),
            jax.ShapeDtypeStruct((NP, 1), jnp.float32),
            jax.ShapeDtypeStruct((NP, 1), jnp.float32),
        ],
    )(x_pad, w, b, degv, dege)


def _scale_kernel(p_ref, s_ref, o_ref):
    o_ref[...] = (p_ref[0] + p_ref[1]) * s_ref[...]


def _combine_scale(parts, scale):
    return pl.pallas_call(
        _scale_kernel,
        grid=(NP // _BROWS,),
        in_specs=[
            pl.BlockSpec((2, _BROWS, D), lambda i: (0, i, 0)),
            pl.BlockSpec((_BROWS, 1), lambda i: (i, 0)),
        ],
        out_specs=pl.BlockSpec((_BROWS, D), lambda i: (i, 0)),
        out_shape=jax.ShapeDtypeStruct((NP, D), jnp.float32),
    )(parts, scale)


def _final_kernel(p_ref, s_ref, o_ref):
    y = (p_ref[0] + p_ref[1]) * s_ref[...]
    o_ref[...] = jnp.maximum(y, 0.0)


_FROWS = 2000


def _final(parts, isdv):
    return pl.pallas_call(
        _final_kernel,
        grid=(N // _FROWS,),
        in_specs=[
            pl.BlockSpec((2, _FROWS, D), lambda i: (0, i, 0)),
            pl.BlockSpec((_FROWS, 1), lambda i: (i, 0)),
        ],
        out_specs=pl.BlockSpec((_FROWS, D), lambda i: (i, 0)),
        out_shape=jax.ShapeDtypeStruct((N, D), jnp.float32),
    )(parts, isdv)


def kernel(X, v_idx, e_idx, W, b):
    pad = jnp.full((NNZ_PAD - NNZ,), N, dtype=jnp.int32)
    v_flat = jnp.concatenate([v_idx, pad])
    e_flat = jnp.concatenate([e_idx, pad])
    v_sm = v_flat.reshape(NW, SGN, SG)
    e_sm = e_flat.reshape(NW, SGN, SG)
    v_dg = v_flat.reshape(NW, GD, GSZ)
    e_dg = e_flat.reshape(NW, GD, GSZ)
    x_pad = jnp.concatenate(
        [X, jnp.zeros((NP - N, D), dtype=jnp.float32)], axis=0)

    degv, dege = _degrees(v_dg, e_dg)
    degv = degv.reshape(NC, NP, 1)
    dege = dege.reshape(NC, NP, 1)
    y, isdv, ide = _project(x_pad, W, b.reshape(1, D), degv, dege)

    edge_parts = _smooth_phase(y, v_sm, e_sm)
    edge_feat = _combine_scale(edge_parts.reshape(NC, NP, D), ide)

    vert_parts = _smooth_phase(edge_feat, e_sm, v_sm)
    return _final(vert_parts.reshape(NC, NP, D), isdv)


# speedup vs baseline: 2.3620x; 2.3620x over previous
import functools

import jax
import jax.numpy as jnp
from jax import lax
from jax.experimental import pallas as pl
from jax.experimental.pallas import tpu as pltpu
from jax.experimental.pallas import tpu_sc as plsc

N = 10000
D = 128
DH = D // 2
NNZ = 320000
NC, NS, L = 2, 16, 16
GSZ = 128
G = 160
NNZ_PAD = NS * G * GSZ
NP = 10240
ROWS_PER_TILE = NP // NS
GD = 80
NBUF = 2
NCHUNK = 2
GC = G // NCHUNK


def _sc_mesh():
    return plsc.VectorSubcoreMesh(core_axis_name="c", subcore_axis_name="s")


def _degrees(v_blk, e_blk):
    @functools.partial(
        pl.kernel,
        out_type=(
            jax.ShapeDtypeStruct((NC, NS, ROWS_PER_TILE), jnp.float32),
            jax.ShapeDtypeStruct((NC, NS, ROWS_PER_TILE), jnp.float32),
        ),
        mesh=_sc_mesh(),
        scratch_types=[
            pltpu.VMEM((GD, GSZ), jnp.int32),
            pltpu.VMEM((GD, GSZ), jnp.int32),
            pltpu.VMEM((GSZ,), jnp.float32),
            pltpu.VMEM((ROWS_PER_TILE,), jnp.float32),
            pltpu.VMEM_SHARED((NP,), jnp.float32),
            pltpu.VMEM_SHARED((NP,), jnp.float32),
        ],
    )
    def k(v_hbm, e_hbm, degv_hbm, dege_hbm, idx_v, idx_e, ones, zbuf,
          accv, acce):
        c = lax.axis_index("c")
        s = lax.axis_index("s")
        wid = s * NC + c

        def fill(i, _):
            ones[pl.ds(i * L, L)] = jnp.ones((L,), jnp.float32)
            return 0

        lax.fori_loop(0, GSZ // L, fill, 0)

        def zfill(i, _):
            zbuf[pl.ds(i * L, L)] = jnp.zeros((L,), jnp.float32)
            return 0

        lax.fori_loop(0, ROWS_PER_TILE // L, zfill, 0)

        base = s * ROWS_PER_TILE
        pltpu.sync_copy(zbuf, accv.at[pl.ds(base, ROWS_PER_TILE)])
        pltpu.sync_copy(zbuf, acce.at[pl.ds(base, ROWS_PER_TILE)])
        pltpu.sync_copy(v_hbm.at[wid], idx_v)
        pltpu.sync_copy(e_hbm.at[wid], idx_e)
        plsc.subcore_barrier()

        def body(g, _):
            pltpu.sync_copy(ones, accv.at[idx_v.at[g]], add=True)
            pltpu.sync_copy(ones, acce.at[idx_e.at[g]], add=True)
            return 0

        lax.fori_loop(0, GD, body, 0)
        plsc.subcore_barrier()

        pltpu.sync_copy(accv.at[pl.ds(base, ROWS_PER_TILE)], degv_hbm.at[c, s])
        pltpu.sync_copy(acce.at[pl.ds(base, ROWS_PER_TILE)], dege_hbm.at[c, s])

    return k(v_blk, e_blk)


def _smooth_phase(table, g_blk, s_blk):
    @functools.partial(
        pl.kernel,
        out_type=jax.ShapeDtypeStruct((NC, NS, ROWS_PER_TILE, DH),
                                      jnp.float32),
        mesh=_sc_mesh(),
        compiler_params=pltpu.CompilerParams(use_tc_tiling_on_sc=False),
        scratch_types=[
            pltpu.VMEM((GC, GSZ), jnp.int32),
            pltpu.VMEM((GC, GSZ), jnp.int32),
            [pltpu.VMEM((GSZ, DH), jnp.float32) for _ in range(NBUF)],
            pltpu.VMEM_SHARED((NP, DH), jnp.float32),
            pltpu.VMEM_SHARED((NP, DH), jnp.float32),
            [pltpu.SemaphoreType.DMA for _ in range(NBUF)],
            [pltpu.SemaphoreType.DMA for _ in range(NBUF)],
        ],
    )
    def k(tab_hbm, g_hbm, s_hbm, out_hbm, idx_g, idx_s, bufs, tab, acc,
          gsem, ssem):
        c = lax.axis_index("c")
        s = lax.axis_index("s")
        base = s * ROWS_PER_TILE

        pltpu.sync_copy(tab_hbm.at[c, pl.ds(base, ROWS_PER_TILE)],
                        tab.at[pl.ds(base, ROWS_PER_TILE)])

        def zrow(i, _):
            r = i // (DH // L)
            col = (i % (DH // L)) * L
            bufs[0][r, pl.ds(col, L)] = jnp.zeros((L,), jnp.float32)
            return 0

        lax.fori_loop(0, GSZ * (DH // L), zrow, 0)

        for z in range(ROWS_PER_TILE // GSZ):
            pltpu.sync_copy(bufs[0], acc.at[pl.ds(base + z * GSZ, GSZ)])

        plsc.subcore_barrier()

        for h in range(NCHUNK):
            pltpu.sync_copy(g_hbm.at[s, pl.ds(h * GC, GC)], idx_g)
            pltpu.sync_copy(s_hbm.at[s, pl.ds(h * GC, GC)], idx_s)

            for b in range(NBUF):
                pltpu.async_copy(tab.at[idx_g.at[b]], bufs[b], gsem[b])

            def body(i, _):
                for b in range(NBUF):
                    g = NBUF * i + b
                    pltpu.make_async_copy(
                        tab.at[idx_g.at[g]], bufs[b], gsem[b]).wait()
                    pltpu.async_copy(
                        bufs[b], acc.at[idx_s.at[g]], ssem[b], add=True)

                @pl.when(i + 1 < GC // NBUF)
                def _():
                    for b in range(NBUF):
                        g = NBUF * (i + 1) + b
                        pltpu.make_async_copy(
                            bufs[b], acc.at[idx_s.at[g - NBUF]],
                            ssem[b]).wait()
                        pltpu.async_copy(
                            tab.at[idx_g.at[g]], bufs[b], gsem[b])

                return 0

            lax.fori_loop(0, GC // NBUF, body, 0)
            for b in range(NBUF):
                pltpu.make_async_copy(
                    bufs[b], acc.at[idx_s.at[GC - NBUF + b]], ssem[b]).wait()

        plsc.subcore_barrier()

        pltpu.sync_copy(acc.at[pl.ds(base, ROWS_PER_TILE)], out_hbm.at[c, s])

    return k(table, g_blk, s_blk)


_BROWS = 1280


def _proj_kernel(x_ref, w_ref, b_ref, dv_ref, de_ref, y_ref, isdv_ref,
                 ide_ref):
    dv = dv_ref[0] + dv_ref[1]
    de = de_ref[0] + de_ref[1]
    isdv = jnp.where(dv > 0, lax.rsqrt(dv), 0.0)
    isdv_ref[...] = isdv
    ide_ref[...] = jnp.where(de > 0, 1.0 / de, 0.0)
    y = jnp.dot(x_ref[...], w_ref[...].T, preferred_element_type=jnp.float32)
    y = (y + b_ref[...]) * isdv
    y_ref[0] = y[:, :DH]
    y_ref[1] = y[:, DH:]


def _project(x_pad, w, b, degv, dege):
    return pl.pallas_call(
        _proj_kernel,
        grid=(NP // _BROWS,),
        in_specs=[
            pl.BlockSpec((_BROWS, D), lambda i: (i, 0)),
            pl.BlockSpec((D, D), lambda i: (0, 0)),
            pl.BlockSpec((1, D), lambda i: (0, 0)),
            pl.BlockSpec((2, _BROWS, 1), lambda i: (0, i, 0)),
            pl.BlockSpec((2, _BROWS, 1), lambda i: (0, i, 0)),
        ],
        out_specs=[
            pl.BlockSpec((2, _BROWS, DH), lambda i: (0, i, 0)),
            pl.BlockSpec((_BROWS, 1), lambda i: (i, 0)),
            pl.BlockSpec((_BROWS, 1), lambda i: (i, 0)),
        ],
        out_shape=[
            jax.ShapeDtypeStruct((NC, NP, DH), jnp.float32),
            jax.ShapeDtypeStruct((NP, 1), jnp.float32),
            jax.ShapeDtypeStruct((NP, 1), jnp.float32),
        ],
    )(x_pad, w, b, degv, dege)


def _scale_kernel(p_ref, s_ref, o_ref):
    o_ref[...] = p_ref[...] * s_ref[...]


def _scale_split(parts, scale):
    return pl.pallas_call(
        _scale_kernel,
        grid=(NP // _BROWS,),
        in_specs=[
            pl.BlockSpec((2, _BROWS, DH), lambda i: (0, i, 0)),
            pl.BlockSpec((_BROWS, 1), lambda i: (i, 0)),
        ],
        out_specs=pl.BlockSpec((2, _BROWS, DH), lambda i: (0, i, 0)),
        out_shape=jax.ShapeDtypeStruct((NC, NP, DH), jnp.float32),
    )(parts, scale)


def _final_kernel(p_ref, s_ref, o_ref):
    y = jnp.concatenate([p_ref[0], p_ref[1]], axis=1) * s_ref[...]
    o_ref[...] = jnp.maximum(y, 0.0)


_FROWS = 2000


def _final(parts, isdv):
    return pl.pallas_call(
        _final_kernel,
        grid=(N // _FROWS,),
        in_specs=[
            pl.BlockSpec((2, _FROWS, DH), lambda i: (0, i, 0)),
            pl.BlockSpec((_FROWS, 1), lambda i: (i, 0)),
        ],
        out_specs=pl.BlockSpec((_FROWS, D), lambda i: (i, 0)),
        out_shape=jax.ShapeDtypeStruct((N, D), jnp.float32),
    )(parts, isdv)


def kernel(X, v_idx, e_idx, W, b):
    pad = jnp.full((NNZ_PAD - NNZ,), N, dtype=jnp.int32)
    v_blk = jnp.concatenate([v_idx, pad]).reshape(NS, G, GSZ)
    e_blk = jnp.concatenate([e_idx, pad]).reshape(NS, G, GSZ)
    v32 = v_blk.reshape(NC * NS, GD, GSZ)
    e32 = e_blk.reshape(NC * NS, GD, GSZ)
    x_pad = jnp.concatenate(
        [X, jnp.zeros((NP - N, D), dtype=jnp.float32)], axis=0)

    degv, dege = _degrees(v32, e32)
    degv = degv.reshape(NC, NP, 1)
    dege = dege.reshape(NC, NP, 1)
    y, isdv, ide = _project(x_pad, W, b.reshape(1, D), degv, dege)

    edge_sums = _smooth_phase(y, v_blk, e_blk)
    edge_feat = _scale_split(edge_sums.reshape(NC, NP, DH), ide)

    vert_sums = _smooth_phase(edge_feat, e_blk, v_blk)
    return _final(vert_sums.reshape(NC, NP, DH), isdv)


# speedup vs baseline: 2.4974x; 1.0573x over previous
import functools

import jax
import jax.numpy as jnp
from jax import lax
from jax.experimental import pallas as pl
from jax.experimental.pallas import tpu as pltpu
from jax.experimental.pallas import tpu_sc as plsc

N = 10000
D = 128
DH = D // 2
NNZ = 320000
NC, NS, L = 2, 16, 16
GSZ = 128
G = 160
NNZ_PAD = NS * G * GSZ
NP = 10240
ROWS_PER_TILE = NP // NS
GD = 80
NBUF = 2
NCHUNK = 2
GC = G // NCHUNK


def _sc_mesh():
    return plsc.VectorSubcoreMesh(core_axis_name="c", subcore_axis_name="s")


def _degrees(v_blk, e_blk):
    @functools.partial(
        pl.kernel,
        out_type=(
            jax.ShapeDtypeStruct((NC, NS, ROWS_PER_TILE), jnp.float32),
            jax.ShapeDtypeStruct((NC, NS, ROWS_PER_TILE), jnp.float32),
        ),
        mesh=_sc_mesh(),
        scratch_types=[
            pltpu.VMEM((GD, GSZ), jnp.int32),
            pltpu.VMEM((GD, GSZ), jnp.int32),
            pltpu.VMEM((GSZ,), jnp.float32),
            pltpu.VMEM((ROWS_PER_TILE,), jnp.float32),
            pltpu.VMEM_SHARED((NP,), jnp.float32),
            pltpu.VMEM_SHARED((NP,), jnp.float32),
        ],
    )
    def k(v_hbm, e_hbm, degv_hbm, dege_hbm, idx_v, idx_e, ones, zbuf,
          accv, acce):
        c = lax.axis_index("c")
        s = lax.axis_index("s")
        wid = s * NC + c

        def fill(i, _):
            ones[pl.ds(i * L, L)] = jnp.ones((L,), jnp.float32)
            return 0

        lax.fori_loop(0, GSZ // L, fill, 0)

        def zfill(i, _):
            zbuf[pl.ds(i * L, L)] = jnp.zeros((L,), jnp.float32)
            return 0

        lax.fori_loop(0, ROWS_PER_TILE // L, zfill, 0)

        base = s * ROWS_PER_TILE
        pltpu.sync_copy(zbuf, accv.at[pl.ds(base, ROWS_PER_TILE)])
        pltpu.sync_copy(zbuf, acce.at[pl.ds(base, ROWS_PER_TILE)])
        pltpu.sync_copy(v_hbm.at[wid], idx_v)
        pltpu.sync_copy(e_hbm.at[wid], idx_e)
        plsc.subcore_barrier()

        def body(g, _):
            pltpu.sync_copy(ones, accv.at[idx_v.at[g]], add=True)
            pltpu.sync_copy(ones, acce.at[idx_e.at[g]], add=True)
            return 0

        lax.fori_loop(0, GD, body, 0)
        plsc.subcore_barrier()

        pltpu.sync_copy(accv.at[pl.ds(base, ROWS_PER_TILE)], degv_hbm.at[c, s])
        pltpu.sync_copy(acce.at[pl.ds(base, ROWS_PER_TILE)], dege_hbm.at[c, s])

    return k(v_blk, e_blk)


def _fused_smooth(y_col, ide, v_blk, e_blk):
    @functools.partial(
        pl.kernel,
        out_type=jax.ShapeDtypeStruct((NC, NS, ROWS_PER_TILE, DH),
                                      jnp.float32),
        mesh=_sc_mesh(),
        compiler_params=pltpu.CompilerParams(use_tc_tiling_on_sc=False),
        scratch_types=[
            pltpu.VMEM((GC, GSZ), jnp.int32),
            pltpu.VMEM((GC, GSZ), jnp.int32),
            [pltpu.VMEM((GSZ, DH), jnp.float32) for _ in range(NBUF)],
            pltpu.VMEM((ROWS_PER_TILE + L,), jnp.float32),
            pltpu.VMEM_SHARED((NP, DH), jnp.float32),
            pltpu.VMEM_SHARED((NP, DH), jnp.float32),
            [pltpu.SemaphoreType.DMA for _ in range(NBUF)],
            [pltpu.SemaphoreType.DMA for _ in range(NBUF)],
        ],
    )
    def k(y_hbm, ide_hbm, v_hbm, e_hbm, out_hbm, idx_g, idx_s, bufs, ide_v,
          tabA, tabB, gsem, ssem):
        c = lax.axis_index("c")
        s = lax.axis_index("s")
        base = s * ROWS_PER_TILE

        def zero_buf(buf):
            def zrow(i, _):
                r = i // (DH // L)
                col = (i % (DH // L)) * L
                buf[r, pl.ds(col, L)] = jnp.zeros((L,), jnp.float32)
                return 0

            lax.fori_loop(0, GSZ * (DH // L), zrow, 0)

        def zero_slice(tab, buf):
            for z in range(ROWS_PER_TILE // GSZ):
                pltpu.sync_copy(buf, tab.at[pl.ds(base + z * GSZ, GSZ)])

        def run_phase(tab_src, tab_dst, g_hbm, s_hbm):
            for h in range(NCHUNK):
                pltpu.sync_copy(g_hbm.at[s, pl.ds(h * GC, GC)], idx_g)
                pltpu.sync_copy(s_hbm.at[s, pl.ds(h * GC, GC)], idx_s)

                for b in range(NBUF):
                    pltpu.async_copy(
                        tab_src.at[idx_g.at[b]], bufs[b], gsem[b])

                def body(i, _):
                    for b in range(NBUF):
                        g = NBUF * i + b
                        pltpu.make_async_copy(
                            tab_src.at[idx_g.at[g]], bufs[b], gsem[b]).wait()
                        pltpu.async_copy(
                            bufs[b], tab_dst.at[idx_s.at[g]], ssem[b],
                            add=True)

                    @pl.when(i + 1 < GC // NBUF)
                    def _():
                        for b in range(NBUF):
                            g = NBUF * (i + 1) + b
                            pltpu.make_async_copy(
                                bufs[b], tab_dst.at[idx_s.at[g - NBUF]],
                                ssem[b]).wait()
                            pltpu.async_copy(
                                tab_src.at[idx_g.at[g]], bufs[b], gsem[b])

                    return 0

                lax.fori_loop(0, GC // NBUF, body, 0)
                for b in range(NBUF):
                    pltpu.make_async_copy(
                        bufs[b], tab_dst.at[idx_s.at[GC - NBUF + b]],
                        ssem[b]).wait()

        pltpu.sync_copy(y_hbm.at[c, pl.ds(base, ROWS_PER_TILE)],
                        tabA.at[pl.ds(base, ROWS_PER_TILE)])
        pltpu.sync_copy(ide_hbm.at[pl.ds(base, ROWS_PER_TILE)],
                        ide_v.at[pl.ds(0, ROWS_PER_TILE)])
        zero_buf(bufs[0])
        zero_slice(tabB, bufs[0])
        plsc.subcore_barrier()

        run_phase(tabA, tabB, v_hbm, e_hbm)
        plsc.subcore_barrier()

        for z in range(ROWS_PER_TILE // GSZ):
            pltpu.sync_copy(tabB.at[pl.ds(base + z * GSZ, GSZ)], bufs[0])

            def scale_row(r, _):
                v = ide_v[pl.ds(z * GSZ + r, L)]
                f = jnp.full((L,), v[0], jnp.float32)
                for col in range(DH // L):
                    sl = pl.ds(col * L, L)
                    bufs[0][r, sl] = bufs[0][r, sl] * f
                return 0

            lax.fori_loop(0, GSZ, scale_row, 0)
            pltpu.sync_copy(bufs[0], tabB.at[pl.ds(base + z * GSZ, GSZ)])

        zero_buf(bufs[0])
        zero_slice(tabA, bufs[0])
        plsc.subcore_barrier()

        run_phase(tabB, tabA, e_hbm, v_hbm)
        plsc.subcore_barrier()

        pltpu.sync_copy(tabA.at[pl.ds(base, ROWS_PER_TILE)], out_hbm.at[c, s])

    return k(y_col, ide, v_blk, e_blk)


_BROWS = 1280


def _proj_kernel(x_ref, w_ref, b_ref, dv_ref, de_ref, y_ref, isdv_ref,
                 ide_ref):
    dv = dv_ref[0] + dv_ref[1]
    de = de_ref[0] + de_ref[1]
    isdv = jnp.where(dv > 0, lax.rsqrt(dv), 0.0)
    isdv_ref[...] = isdv
    ide_ref[...] = jnp.where(de > 0, 1.0 / de, 0.0)
    y = jnp.dot(x_ref[...], w_ref[...].T, preferred_element_type=jnp.float32)
    y = (y + b_ref[...]) * isdv
    y_ref[0] = y[:, :DH]
    y_ref[1] = y[:, DH:]


def _project(x_pad, w, b, degv, dege):
    return pl.pallas_call(
        _proj_kernel,
        grid=(NP // _BROWS,),
        in_specs=[
            pl.BlockSpec((_BROWS, D), lambda i: (i, 0)),
            pl.BlockSpec((D, D), lambda i: (0, 0)),
            pl.BlockSpec((1, D), lambda i: (0, 0)),
            pl.BlockSpec((2, _BROWS, 1), lambda i: (0, i, 0)),
            pl.BlockSpec((2, _BROWS, 1), lambda i: (0, i, 0)),
        ],
        out_specs=[
            pl.BlockSpec((2, _BROWS, DH), lambda i: (0, i, 0)),
            pl.BlockSpec((_BROWS, 1), lambda i: (i, 0)),
            pl.BlockSpec((_BROWS, 1), lambda i: (i, 0)),
        ],
        out_shape=[
            jax.ShapeDtypeStruct((NC, NP, DH), jnp.float32),
            jax.ShapeDtypeStruct((NP, 1), jnp.float32),
            jax.ShapeDtypeStruct((NP, 1), jnp.float32),
        ],
    )(x_pad, w, b, degv, dege)


def _scale_kernel(p_ref, s_ref, o_ref):
    o_ref[...] = p_ref[...] * s_ref[...]


def _scale_split(parts, scale):
    return pl.pallas_call(
        _scale_kernel,
        grid=(NP // _BROWS,),
        in_specs=[
            pl.BlockSpec((2, _BROWS, DH), lambda i: (0, i, 0)),
            pl.BlockSpec((_BROWS, 1), lambda i: (i, 0)),
        ],
        out_specs=pl.BlockSpec((2, _BROWS, DH), lambda i: (0, i, 0)),
        out_shape=jax.ShapeDtypeStruct((NC, NP, DH), jnp.float32),
    )(parts, scale)


def _final_kernel(p_ref, s_ref, o_ref):
    y = jnp.concatenate([p_ref[0], p_ref[1]], axis=1) * s_ref[...]
    o_ref[...] = jnp.maximum(y, 0.0)


_FROWS = 2000


def _final(parts, isdv):
    return pl.pallas_call(
        _final_kernel,
        grid=(N // _FROWS,),
        in_specs=[
            pl.BlockSpec((2, _FROWS, DH), lambda i: (0, i, 0)),
            pl.BlockSpec((_FROWS, 1), lambda i: (i, 0)),
        ],
        out_specs=pl.BlockSpec((_FROWS, D), lambda i: (i, 0)),
        out_shape=jax.ShapeDtypeStruct((N, D), jnp.float32),
    )(parts, isdv)


def kernel(X, v_idx, e_idx, W, b):
    pad = jnp.full((NNZ_PAD - NNZ,), N, dtype=jnp.int32)
    v_blk = jnp.concatenate([v_idx, pad]).reshape(NS, G, GSZ)
    e_blk = jnp.concatenate([e_idx, pad]).reshape(NS, G, GSZ)
    v32 = v_blk.reshape(NC * NS, GD, GSZ)
    e32 = e_blk.reshape(NC * NS, GD, GSZ)
    x_pad = jnp.concatenate(
        [X, jnp.zeros((NP - N, D), dtype=jnp.float32)], axis=0)

    degv, dege = _degrees(v32, e32)
    degv = degv.reshape(NC, NP, 1)
    dege = dege.reshape(NC, NP, 1)
    y, isdv, ide = _project(x_pad, W, b.reshape(1, D), degv, dege)

    vert_sums = _fused_smooth(y, ide.reshape(NP), v_blk, e_blk)
    return _final(vert_sums.reshape(NC, NP, DH), isdv)


# speedup vs baseline: 2.5385x; 1.0164x over previous
import functools

import jax
import jax.numpy as jnp
from jax import lax
from jax.experimental import pallas as pl
from jax.experimental.pallas import tpu as pltpu
from jax.experimental.pallas import tpu_sc as plsc

N = 10000
D = 128
DH = D // 2
NNZ = 320000
NC, NS, L = 2, 16, 16
GSZ = 128
G = 160
NNZ_PAD = NS * G * GSZ
NP = 10240
ROWS_PER_TILE = NP // NS
GD = 80
NBUF = 2
NCHUNK = 2
GC = G // NCHUNK


def _sc_mesh():
    return plsc.VectorSubcoreMesh(core_axis_name="c", subcore_axis_name="s")


def _degrees(v_blk, e_blk):
    @functools.partial(
        pl.kernel,
        out_type=(
            jax.ShapeDtypeStruct((NC, NS, ROWS_PER_TILE), jnp.float32),
            jax.ShapeDtypeStruct((NC, NS, ROWS_PER_TILE), jnp.float32),
        ),
        mesh=_sc_mesh(),
        scratch_types=[
            pltpu.VMEM((GD, GSZ), jnp.int32),
            pltpu.VMEM((GD, GSZ), jnp.int32),
            pltpu.VMEM((GSZ,), jnp.float32),
            pltpu.VMEM((ROWS_PER_TILE,), jnp.float32),
            pltpu.VMEM_SHARED((NP,), jnp.float32),
            pltpu.VMEM_SHARED((NP,), jnp.float32),
        ],
    )
    def k(v_hbm, e_hbm, degv_hbm, dege_hbm, idx_v, idx_e, ones, zbuf,
          accv, acce):
        c = lax.axis_index("c")
        s = lax.axis_index("s")
        wid = s * NC + c

        def fill(i, _):
            ones[pl.ds(i * L, L)] = jnp.ones((L,), jnp.float32)
            return 0

        lax.fori_loop(0, GSZ // L, fill, 0)

        def zfill(i, _):
            zbuf[pl.ds(i * L, L)] = jnp.zeros((L,), jnp.float32)
            return 0

        lax.fori_loop(0, ROWS_PER_TILE // L, zfill, 0)

        base = s * ROWS_PER_TILE
        pltpu.sync_copy(zbuf, accv.at[pl.ds(base, ROWS_PER_TILE)])
        pltpu.sync_copy(zbuf, acce.at[pl.ds(base, ROWS_PER_TILE)])
        pltpu.sync_copy(v_hbm.at[wid], idx_v)
        pltpu.sync_copy(e_hbm.at[wid], idx_e)
        plsc.subcore_barrier()

        def body(g, _):
            pltpu.sync_copy(ones, accv.at[idx_v.at[g]], add=True)
            pltpu.sync_copy(ones, acce.at[idx_e.at[g]], add=True)
            return 0

        lax.fori_loop(0, GD, body, 0)
        plsc.subcore_barrier()

        pltpu.sync_copy(accv.at[pl.ds(base, ROWS_PER_TILE)], degv_hbm.at[c, s])
        pltpu.sync_copy(acce.at[pl.ds(base, ROWS_PER_TILE)], dege_hbm.at[c, s])

    return k(v_blk, e_blk)


def _fused_smooth(y_col, ide, isdv, v_blk, e_blk):
    @functools.partial(
        pl.kernel,
        out_type=jax.ShapeDtypeStruct((NP, D), jnp.float32),
        mesh=_sc_mesh(),
        compiler_params=pltpu.CompilerParams(use_tc_tiling_on_sc=False),
        scratch_types=[
            pltpu.VMEM((GC, GSZ), jnp.int32),
            pltpu.VMEM((GC, GSZ), jnp.int32),
            [pltpu.VMEM((GSZ, DH), jnp.float32) for _ in range(NBUF)],
            pltpu.VMEM((ROWS_PER_TILE + L,), jnp.float32),
            pltpu.VMEM((ROWS_PER_TILE + L,), jnp.float32),
            pltpu.VMEM_SHARED((NP, DH), jnp.float32),
            pltpu.VMEM_SHARED((NP, DH), jnp.float32),
            [pltpu.SemaphoreType.DMA for _ in range(NBUF)],
            [pltpu.SemaphoreType.DMA for _ in range(NBUF)],
        ],
    )
    def k(y_hbm, ide_hbm, isdv_hbm, v_hbm, e_hbm, out_hbm, idx_g, idx_s,
          bufs, ide_v, isdv_v, tabA, tabB, gsem, ssem):
        c = lax.axis_index("c")
        s = lax.axis_index("s")
        base = s * ROWS_PER_TILE

        def zero_buf(buf):
            def zrow(i, _):
                r = i // (DH // L)
                col = (i % (DH // L)) * L
                buf[r, pl.ds(col, L)] = jnp.zeros((L,), jnp.float32)
                return 0

            lax.fori_loop(0, GSZ * (DH // L), zrow, 0)

        def zero_slice(tab, buf):
            for z in range(ROWS_PER_TILE // GSZ):
                pltpu.sync_copy(buf, tab.at[pl.ds(base + z * GSZ, GSZ)])

        def run_phase(tab_src, tab_dst, g_hbm, s_hbm):
            for h in range(NCHUNK):
                pltpu.sync_copy(g_hbm.at[s, pl.ds(h * GC, GC)], idx_g)
                pltpu.sync_copy(s_hbm.at[s, pl.ds(h * GC, GC)], idx_s)

                for b in range(NBUF):
                    pltpu.async_copy(
                        tab_src.at[idx_g.at[b]], bufs[b], gsem[b])

                def body(i, _):
                    for b in range(NBUF):
                        g = NBUF * i + b
                        pltpu.make_async_copy(
                            tab_src.at[idx_g.at[g]], bufs[b], gsem[b]).wait()
                        pltpu.async_copy(
                            bufs[b], tab_dst.at[idx_s.at[g]], ssem[b],
                            add=True)

                    @pl.when(i + 1 < GC // NBUF)
                    def _():
                        for b in range(NBUF):
                            g = NBUF * (i + 1) + b
                            pltpu.make_async_copy(
                                bufs[b], tab_dst.at[idx_s.at[g - NBUF]],
                                ssem[b]).wait()
                            pltpu.async_copy(
                                tab_src.at[idx_g.at[g]], bufs[b], gsem[b])

                    return 0

                lax.fori_loop(0, GC // NBUF, body, 0)
                for b in range(NBUF):
                    pltpu.make_async_copy(
                        bufs[b], tab_dst.at[idx_s.at[GC - NBUF + b]],
                        ssem[b]).wait()

        pltpu.sync_copy(y_hbm.at[c, pl.ds(base, ROWS_PER_TILE)],
                        tabA.at[pl.ds(base, ROWS_PER_TILE)])
        pltpu.sync_copy(ide_hbm.at[pl.ds(base, ROWS_PER_TILE)],
                        ide_v.at[pl.ds(0, ROWS_PER_TILE)])
        pltpu.sync_copy(isdv_hbm.at[pl.ds(base, ROWS_PER_TILE)],
                        isdv_v.at[pl.ds(0, ROWS_PER_TILE)])
        zero_buf(bufs[0])
        zero_slice(tabB, bufs[0])
        plsc.subcore_barrier()

        run_phase(tabA, tabB, v_hbm, e_hbm)
        plsc.subcore_barrier()

        for z in range(ROWS_PER_TILE // GSZ):
            pltpu.sync_copy(tabB.at[pl.ds(base + z * GSZ, GSZ)], bufs[0])

            def scale_row(r, _):
                v = ide_v[pl.ds(z * GSZ + r, L)]
                f = jnp.full((L,), v[0], jnp.float32)
                for col in range(DH // L):
                    sl = pl.ds(col * L, L)
                    bufs[0][r, sl] = bufs[0][r, sl] * f
                return 0

            lax.fori_loop(0, GSZ, scale_row, 0)
            pltpu.sync_copy(bufs[0], tabB.at[pl.ds(base + z * GSZ, GSZ)])

        zero_buf(bufs[0])
        zero_slice(tabA, bufs[0])
        plsc.subcore_barrier()

        run_phase(tabB, tabA, e_hbm, v_hbm)
        plsc.subcore_barrier()

        for z in range(ROWS_PER_TILE // GSZ):
            pltpu.sync_copy(tabA.at[pl.ds(base + z * GSZ, GSZ)], bufs[0])

            def out_row(r, _):
                v = isdv_v[pl.ds(z * GSZ + r, L)]
                f = jnp.full((L,), v[0], jnp.float32)
                zero = jnp.zeros((L,), jnp.float32)
                for col in range(DH // L):
                    sl = pl.ds(col * L, L)
                    bufs[0][r, sl] = jnp.maximum(bufs[0][r, sl] * f, zero)
                return 0

            lax.fori_loop(0, GSZ, out_row, 0)
            pltpu.sync_copy(
                bufs[0],
                out_hbm.at[pl.ds(base + z * GSZ, GSZ), pl.ds(c * DH, DH)])

    return k(y_col, ide, isdv, v_blk, e_blk)


_BROWS = 1280


def _proj_kernel(x_ref, w_ref, b_ref, dv_ref, de_ref, y_ref, isdv_ref,
                 ide_ref):
    dv = dv_ref[0] + dv_ref[1]
    de = de_ref[0] + de_ref[1]
    isdv = jnp.where(dv > 0, lax.rsqrt(dv), 0.0)
    isdv_ref[...] = isdv
    ide_ref[...] = jnp.where(de > 0, 1.0 / de, 0.0)
    y = jnp.dot(x_ref[...], w_ref[...].T, preferred_element_type=jnp.float32)
    y = (y + b_ref[...]) * isdv
    y_ref[0] = y[:, :DH]
    y_ref[1] = y[:, DH:]


def _project(x_pad, w, b, degv, dege):
    return pl.pallas_call(
        _proj_kernel,
        grid=(NP // _BROWS,),
        in_specs=[
            pl.BlockSpec((_BROWS, D), lambda i: (i, 0)),
            pl.BlockSpec((D, D), lambda i: (0, 0)),
            pl.BlockSpec((1, D), lambda i: (0, 0)),
            pl.BlockSpec((2, _BROWS, 1), lambda i: (0, i, 0)),
            pl.BlockSpec((2, _BROWS, 1), lambda i: (0, i, 0)),
        ],
        out_specs=[
            pl.BlockSpec((2, _BROWS, DH), lambda i: (0, i, 0)),
            pl.BlockSpec((_BROWS, 1), lambda i: (i, 0)),
            pl.BlockSpec((_BROWS, 1), lambda i: (i, 0)),
        ],
        out_shape=[
            jax.ShapeDtypeStruct((NC, NP, DH), jnp.float32),
            jax.ShapeDtypeStruct((NP, 1), jnp.float32),
            jax.ShapeDtypeStruct((NP, 1), jnp.float32),
        ],
    )(x_pad, w, b, degv, dege)


def _scale_kernel(p_ref, s_ref, o_ref):
    o_ref[...] = p_ref[...] * s_ref[...]


def _scale_split(parts, scale):
    return pl.pallas_call(
        _scale_kernel,
        grid=(NP // _BROWS,),
        in_specs=[
            pl.BlockSpec((2, _BROWS, DH), lambda i: (0, i, 0)),
            pl.BlockSpec((_BROWS, 1), lambda i: (i, 0)),
        ],
        out_specs=pl.BlockSpec((2, _BROWS, DH), lambda i: (0, i, 0)),
        out_shape=jax.ShapeDtypeStruct((NC, NP, DH), jnp.float32),
    )(parts, scale)


def _final_kernel(p_ref, s_ref, o_ref):
    y = jnp.concatenate([p_ref[0], p_ref[1]], axis=1) * s_ref[...]
    o_ref[...] = jnp.maximum(y, 0.0)


_FROWS = 2000


def _final(parts, isdv):
    return pl.pallas_call(
        _final_kernel,
        grid=(N // _FROWS,),
        in_specs=[
            pl.BlockSpec((2, _FROWS, DH), lambda i: (0, i, 0)),
            pl.BlockSpec((_FROWS, 1), lambda i: (i, 0)),
        ],
        out_specs=pl.BlockSpec((_FROWS, D), lambda i: (i, 0)),
        out_shape=jax.ShapeDtypeStruct((N, D), jnp.float32),
    )(parts, isdv)


def kernel(X, v_idx, e_idx, W, b):
    pad = jnp.full((NNZ_PAD - NNZ,), N, dtype=jnp.int32)
    v_blk = jnp.concatenate([v_idx, pad]).reshape(NS, G, GSZ)
    e_blk = jnp.concatenate([e_idx, pad]).reshape(NS, G, GSZ)
    v32 = v_blk.reshape(NC * NS, GD, GSZ)
    e32 = e_blk.reshape(NC * NS, GD, GSZ)
    x_pad = jnp.concatenate(
        [X, jnp.zeros((NP - N, D), dtype=jnp.float32)], axis=0)

    degv, dege = _degrees(v32, e32)
    degv = degv.reshape(NC, NP, 1)
    dege = dege.reshape(NC, NP, 1)
    y, isdv, ide = _project(x_pad, W, b.reshape(1, D), degv, dege)

    out = _fused_smooth(y, ide.reshape(NP), isdv.reshape(NP), v_blk, e_blk)
    return out[:N]
